# baseline (speedup 1.0000x reference)
import jax
import jax.numpy as jnp
from jax import lax
from jax.experimental import pallas as pl
from jax.experimental.pallas import tpu as pltpu
from jax.experimental.pallas import tpu_sc as plsc

N = 10000
E = 320000
D = 128
H = 64
NPAD = 10240
CHUNK = 128
NCHUNK = E // CHUNK
CH_PER_CORE = NCHUNK // 2
CH_BASE = CH_PER_CORE // 16
CH_REM = CH_PER_CORE - CH_BASE * 16
RPT = NPAD // 16
HOPS = RPT // CHUNK
BN = 512
GRID = NPAD // BN

f32 = jnp.float32
i32 = jnp.int32

_mesh = plsc.VectorSubcoreMesh(core_axis_name="c", subcore_axis_name="s")
_sc_params = pltpu.CompilerParams(use_tc_tiling_on_sc=False)


def _chunk_range(c, s):
    ntch = CH_BASE + jnp.where(s < CH_REM, 1, 0)
    start = c * CH_PER_CORE + s * CH_BASE + jnp.minimum(s, CH_REM)
    return start, ntch


def _hist_body(col1, col2, col3, ew1, ew2, ew3, P,
               colbuf0, colbuf1, valbuf0, valbuf1, onesbuf, tmp, zbuf,
               wd0, cn0, wd1, cn1, wd2, cn2,
               ldsem0, ldsem1, sca0, sca1, scb0, scb1):
    c = lax.axis_index("c")
    s = lax.axis_index("s")
    zero16 = jnp.zeros((16,), f32)
    one16 = jnp.ones((16,), f32)
    colbuf = (colbuf0, colbuf1)
    valbuf = (valbuf0, valbuf1)
    ldsem = (ldsem0, ldsem1)
    sca = (sca0, sca1)
    scb = (scb0, scb1)
    for i in range(RPT // 16):
        zbuf[pl.ds(i * 16, 16)] = zero16
    for i in range(CHUNK // 16):
        onesbuf[pl.ds(i * 16, 16)] = one16
    sl = pl.ds(s * RPT, RPT)
    for arr in (wd0, cn0, wd1, cn1, wd2, cn2):
        pltpu.sync_copy(zbuf, arr.at[sl])
    plsc.subcore_barrier()
    start, ntch = _chunk_range(c, s)
    end = start + ntch
    for colr, ewr, wd, cn in ((col1, ew1, wd0, cn0),
                              (col2, ew2, wd1, cn1),
                              (col3, ew3, wd2, cn2)):
        def fire_loads(cc, b, colr=colr, ewr=ewr):
            base = cc * CHUNK
            pltpu.async_copy(colr.at[pl.ds(base, CHUNK)], colbuf[b], ldsem[b])
            pltpu.async_copy(ewr.at[pl.ds(base, CHUNK)], valbuf[b], ldsem[b])

        def wait_scatters(b, wd=wd, cn=cn):
            pltpu.make_async_copy(valbuf[b], wd.at[colbuf[b]], sca[b]).wait()
            pltpu.make_async_copy(onesbuf, cn.at[colbuf[b]], scb[b]).wait()

        def proc(cc, b, colr=colr, ewr=ewr, wd=wd, cn=cn):
            @pl.when(cc >= start + 1)
            def _():
                wait_scatters(1 - b)

            @pl.when(cc + 1 < end)
            def _():
                fire_loads(cc + 1, 1 - b)
            pltpu.make_async_copy(colr.at[pl.ds(0, CHUNK)], colbuf[b],
                                  ldsem[b]).wait()
            pltpu.make_async_copy(ewr.at[pl.ds(0, CHUNK)], valbuf[b],
                                  ldsem[b]).wait()
            pltpu.async_copy(valbuf[b], wd.at[colbuf[b]], sca[b], add=True)
            pltpu.async_copy(onesbuf, cn.at[colbuf[b]], scb[b], add=True)

        fire_loads(start, 0)

        def pair(k, carry):
            proc(start + 2 * k, 0)
            proc(start + 2 * k + 1, 1)
            return carry
        lax.fori_loop(0, ntch // 2, pair, 0)

        @pl.when(ntch % 2 == 1)
        def _(wait_scatters=wait_scatters, proc=proc):
            proc(end - 1, 0)
            wait_scatters(0)

        @pl.when(ntch % 2 == 0)
        def _(wait_scatters=wait_scatters):
            wait_scatters(1)
    plsc.subcore_barrier()
    for k, arr in enumerate((wd0, wd1, wd2, cn0, cn1, cn2)):
        pltpu.sync_copy(arr.at[sl], tmp)
        pltpu.sync_copy(tmp, P.at[pl.ds((c * 6 + k) * NPAD + s * RPT, RPT)])


_hist = pl.kernel(
    _hist_body,
    out_type=jax.ShapeDtypeStruct((12 * NPAD,), f32),
    mesh=_mesh,
    compiler_params=_sc_params,
    scratch_types=[
        pltpu.VMEM((CHUNK,), i32),
        pltpu.VMEM((CHUNK,), i32),
        pltpu.VMEM((CHUNK,), f32),
        pltpu.VMEM((CHUNK,), f32),
        pltpu.VMEM((CHUNK,), f32),
        pltpu.VMEM((RPT,), f32),
        pltpu.VMEM((RPT,), f32),
    ] + [pltpu.VMEM_SHARED((NPAD,), f32)] * 6
      + [pltpu.SemaphoreType.DMA] * 6,
)


def _agg_body(g1, g2, g3, row1, row2, row3, col1, col2, col3,
              ew1, ew2, ew3, accp,
              rowbuf0, rowbuf1, colbuf0, colbuf1, ewbuf0, ewbuf1,
              rows0, rows1, zrows, acc,
              ldsem0, ldsem1, gsem, scsem0, scsem1):
    c = lax.axis_index("c")
    s = lax.axis_index("s")
    zero16 = jnp.zeros((16,), f32)
    rowbuf = (rowbuf0, rowbuf1)
    colbuf = (colbuf0, colbuf1)
    ewbuf = (ewbuf0, ewbuf1)
    rows = (rows0, rows1)
    ldsem = (ldsem0, ldsem1)
    scsem = (scsem0, scsem1)

    for r in range(CHUNK):
        for q in range(H // 16):
            zrows[r, pl.ds(q * 16, 16)] = zero16
    start, ntch = _chunk_range(c, s)
    end = start + ntch
    for v, (gv, rowr, colr, ewr) in enumerate(((g1, row1, col1, ew1),
                                               (g2, row2, col2, ew2),
                                               (g3, row3, col3, ew3))):
        for hp in range(HOPS):
            r0 = s * RPT + hp * CHUNK
            pltpu.sync_copy(zrows, acc.at[pl.ds(r0, CHUNK)])
        plsc.subcore_barrier()

        def fire_loads(cc, b, rowr=rowr, colr=colr, ewr=ewr):
            base = cc * CHUNK
            pltpu.async_copy(rowr.at[pl.ds(base, CHUNK)], rowbuf[b], ldsem[b])
            pltpu.async_copy(colr.at[pl.ds(base, CHUNK)], colbuf[b], ldsem[b])
            pltpu.async_copy(ewr.at[pl.ds(base, CHUNK)], ewbuf[b], ldsem[b])

        def wait_loads(b, rowr=rowr, colr=colr, ewr=ewr):
            pltpu.make_async_copy(rowr.at[pl.ds(0, CHUNK)], rowbuf[b],
                                  ldsem[b]).wait()
            pltpu.make_async_copy(colr.at[pl.ds(0, CHUNK)], colbuf[b],
                                  ldsem[b]).wait()
            pltpu.make_async_copy(ewr.at[pl.ds(0, CHUNK)], ewbuf[b],
                                  ldsem[b]).wait()

        def wait_scatter(b):
            pltpu.make_async_copy(rows[b], acc.at[colbuf[b]],
                                  scsem[b]).wait()

        def proc(cc, b, gv=gv, rowr=rowr, colr=colr, ewr=ewr):
            @pl.when(cc >= start + 1)
            def _():
                wait_scatter(1 - b)

            @pl.when(cc + 1 < end)
            def _():
                fire_loads(cc + 1, 1 - b)
            wait_loads(b)
            pltpu.async_copy(gv.at[rowbuf[b]], rows[b], gsem).wait()

            def scale(gg, cr):
                vew = ewbuf[b][pl.ds(gg * 16, 16)]
                for eo in range(16):
                    w = vew.at[jnp.full((16,), eo, i32)].get(
                        mode="promise_in_bounds")
                    e = gg * 16 + eo
                    for q in range(H // 16):
                        rows[b][e, pl.ds(q * 16, 16)] = (
                            rows[b][e, pl.ds(q * 16, 16)] * w)
                return cr
            lax.fori_loop(0, CHUNK // 16, scale, 0)
            pltpu.async_copy(rows[b], acc.at[colbuf[b]], scsem[b], add=True)

        fire_loads(start, 0)

        def pair(k, carry):
            proc(start + 2 * k, 0)
            proc(start + 2 * k + 1, 1)
            return carry
        lax.fori_loop(0, ntch // 2, pair, 0)

        @pl.when(ntch % 2 == 1)
        def _():
            proc(end - 1, 0)
            wait_scatter(0)

        @pl.when(ntch % 2 == 0)
        def _():
            wait_scatter(1)
        plsc.subcore_barrier()
        for hp in range(HOPS):
            r0 = s * RPT + hp * CHUNK
            pltpu.sync_copy(acc.at[pl.ds(r0, CHUNK)], rows0)
            pltpu.sync_copy(rows0, accp.at[c, v, pl.ds(r0, CHUNK)])
        plsc.subcore_barrier()


_agg = pl.kernel(
    _agg_body,
    out_type=jax.ShapeDtypeStruct((2, 3, NPAD, H), f32),
    mesh=_mesh,
    compiler_params=_sc_params,
    scratch_types=[
        pltpu.VMEM((CHUNK,), i32),
        pltpu.VMEM((CHUNK,), i32),
        pltpu.VMEM((CHUNK,), i32),
        pltpu.VMEM((CHUNK,), i32),
        pltpu.VMEM((CHUNK,), f32),
        pltpu.VMEM((CHUNK,), f32),
        pltpu.VMEM((CHUNK, H), f32),
        pltpu.VMEM((CHUNK, H), f32),
        pltpu.VMEM((CHUNK, H), f32),
        pltpu.VMEM_SHARED((NPAD, H), f32),
        pltpu.SemaphoreType.DMA,
        pltpu.SemaphoreType.DMA,
        pltpu.SemaphoreType.DMA,
        pltpu.SemaphoreType.DMA,
        pltpu.SemaphoreType.DMA,
    ],
)


def _l2_body(g2, row1, row2, row3, col1, col2, col3, a2p,
             rowbuf0, rowbuf1, colbuf0, colbuf1, valbuf0, valbuf1,
             tmp, zbuf, t0, a0, t1, a1, t2, a2,
             ldsem0, ldsem1, gsem, scsem0, scsem1):
    c = lax.axis_index("c")
    s = lax.axis_index("s")
    zero16 = jnp.zeros((16,), f32)
    rowbuf = (rowbuf0, rowbuf1)
    colbuf = (colbuf0, colbuf1)
    valbuf = (valbuf0, valbuf1)
    ldsem = (ldsem0, ldsem1)
    scsem = (scsem0, scsem1)
    for i in range(RPT // 16):
        zbuf[pl.ds(i * 16, 16)] = zero16
    sl = pl.ds(s * RPT, RPT)
    for v, (tv, av) in enumerate(((t0, a0), (t1, a1), (t2, a2))):
        pltpu.sync_copy(g2.at[pl.ds(v * NPAD + s * RPT, RPT)], tmp)
        pltpu.sync_copy(tmp, tv.at[sl])
        pltpu.sync_copy(zbuf, av.at[sl])
    plsc.subcore_barrier()
    start, ntch = _chunk_range(c, s)
    end = start + ntch
    for (rowr, colr, tv, av) in ((row1, col1, t0, a0),
                                 (row2, col2, t1, a1),
                                 (row3, col3, t2, a2)):
        def fire_loads(cc, b, rowr=rowr, colr=colr):
            base = cc * CHUNK
            pltpu.async_copy(rowr.at[pl.ds(base, CHUNK)], rowbuf[b], ldsem[b])
            pltpu.async_copy(colr.at[pl.ds(base, CHUNK)], colbuf[b], ldsem[b])

        def wait_scatter(b, av=av):
            pltpu.make_async_copy(valbuf[b], av.at[colbuf[b]],
                                  scsem[b]).wait()

        def proc(cc, b, rowr=rowr, colr=colr, tv=tv, av=av):
            @pl.when(cc >= start + 1)
            def _():
                wait_scatter(1 - b)

            @pl.when(cc + 1 < end)
            def _():
                fire_loads(cc + 1, 1 - b)
            pltpu.make_async_copy(rowr.at[pl.ds(0, CHUNK)], rowbuf[b],
                                  ldsem[b]).wait()
            pltpu.make_async_copy(colr.at[pl.ds(0, CHUNK)], colbuf[b],
                                  ldsem[b]).wait()
            pltpu.async_copy(tv.at[rowbuf[b]], valbuf[b], gsem).wait()
            pltpu.async_copy(valbuf[b], av.at[colbuf[b]], scsem[b], add=True)

        fire_loads(start, 0)

        def pair(k, carry):
            proc(start + 2 * k, 0)
            proc(start + 2 * k + 1, 1)
            return carry
        lax.fori_loop(0, ntch // 2, pair, 0)

        @pl.when(ntch % 2 == 1)
        def _(wait_scatter=wait_scatter, proc=proc):
            proc(end - 1, 0)
            wait_scatter(0)

        @pl.when(ntch % 2 == 0)
        def _(wait_scatter=wait_scatter):
            wait_scatter(1)
    plsc.subcore_barrier()
    for v, av in enumerate((a0, a1, a2)):
        pltpu.sync_copy(av.at[sl], tmp)
        pltpu.sync_copy(tmp, a2p.at[pl.ds((c * 3 + v) * NPAD + s * RPT, RPT)])


_l2 = pl.kernel(
    _l2_body,
    out_type=jax.ShapeDtypeStruct((6 * NPAD,), f32),
    mesh=_mesh,
    compiler_params=_sc_params,
    scratch_types=[
        pltpu.VMEM((CHUNK,), i32),
        pltpu.VMEM((CHUNK,), i32),
        pltpu.VMEM((CHUNK,), i32),
        pltpu.VMEM((CHUNK,), i32),
        pltpu.VMEM((CHUNK,), f32),
        pltpu.VMEM((CHUNK,), f32),
        pltpu.VMEM((RPT,), f32),
        pltpu.VMEM((RPT,), f32),
    ] + [pltpu.VMEM_SHARED((NPAD,), f32)] * 6
      + [pltpu.SemaphoreType.DMA] * 5,
)


def _tcb_body(x_ref, w_ref, p_ref, g1_ref, g2_ref, g3_ref, d1_ref, d2_ref):
    h = jnp.dot(x_ref[...], w_ref[...], preferred_element_type=f32)
    p = p_ref[...]
    for v, g_ref in enumerate((g1_ref, g2_ref, g3_ref)):
        d1 = lax.rsqrt(p[0, v, :] + p[1, v, :] + 1.0)
        d2 = lax.rsqrt(p[0, 3 + v, :] + p[1, 3 + v, :] + 1.0)
        d1_ref[v, :] = d1
        d2_ref[v, :] = d2
        g_ref[...] = h[:, v * H:(v + 1) * H] * d1[:, None]


_tcb = pl.pallas_call(
    _tcb_body,
    grid=(GRID,),
    in_specs=[
        pl.BlockSpec((BN, D), lambda i: (i, 0)),
        pl.BlockSpec((D, 3 * H), lambda i: (0, 0)),
        pl.BlockSpec((2, 6, BN), lambda i: (0, 0, i)),
    ],
    out_specs=[
        pl.BlockSpec((BN, H), lambda i: (i, 0)),
        pl.BlockSpec((BN, H), lambda i: (i, 0)),
        pl.BlockSpec((BN, H), lambda i: (i, 0)),
        pl.BlockSpec((3, BN), lambda i: (0, i)),
        pl.BlockSpec((3, BN), lambda i: (0, i)),
    ],
    out_shape=[
        jax.ShapeDtypeStruct((NPAD, H), f32),
        jax.ShapeDtypeStruct((NPAD, H), f32),
        jax.ShapeDtypeStruct((NPAD, H), f32),
        jax.ShapeDtypeStruct((3, NPAD), f32),
        jax.ShapeDtypeStruct((3, NPAD), f32),
    ],
)


def _tcd_body(ap_ref, g1_ref, g2t_ref, g3_ref, d1_ref, d2_ref, b1_ref, w2_ref,
              feat_ref, g2_ref):
    feat = jnp.zeros((BN, H), f32)
    w2 = w2_ref[...]
    for v, g_ref in enumerate((g1_ref, g2t_ref, g3_ref)):
        acc = ap_ref[0, v] + ap_ref[1, v]
        xv = jnp.maximum(
            d1_ref[v][:, None] * (acc + g_ref[...]) + b1_ref[v][None, :], 0.0)
        feat = feat + xv
        h2 = jnp.dot(xv, w2[:, v:v + 1], preferred_element_type=f32)[:, 0]
        g2_ref[v, :] = d2_ref[v] * h2
    feat_ref[...] = feat


_tcd = pl.pallas_call(
    _tcd_body,
    grid=(GRID,),
    in_specs=[
        pl.BlockSpec((2, 3, BN, H), lambda i: (0, 0, i, 0)),
        pl.BlockSpec((BN, H), lambda i: (i, 0)),
        pl.BlockSpec((BN, H), lambda i: (i, 0)),
        pl.BlockSpec((BN, H), lambda i: (i, 0)),
        pl.BlockSpec((3, BN), lambda i: (0, i)),
        pl.BlockSpec((3, BN), lambda i: (0, i)),
        pl.BlockSpec((3, H), lambda i: (0, 0)),
        pl.BlockSpec((H, 3), lambda i: (0, 0)),
    ],
    out_specs=[
        pl.BlockSpec((BN, H), lambda i: (i, 0)),
        pl.BlockSpec((3, BN), lambda i: (0, i)),
    ],
    out_shape=[
        jax.ShapeDtypeStruct((NPAD, H), f32),
        jax.ShapeDtypeStruct((3, NPAD), f32),
    ],
)


def _tcf_body(a2_ref, g2_ref, d2_ref, b2_ref, out_ref):
    b2 = b2_ref[...]
    tot = jnp.zeros((BN,), f32)
    for v in range(3):
        tot = tot + d2_ref[v] * (a2_ref[0, v] + a2_ref[1, v] + g2_ref[v]) \
            + b2[v, 0]
    out_ref[...] = tot[:, None]


_tcf = pl.pallas_call(
    _tcf_body,
    grid=(GRID,),
    in_specs=[
        pl.BlockSpec((2, 3, BN), lambda i: (0, 0, i)),
        pl.BlockSpec((3, BN), lambda i: (0, i)),
        pl.BlockSpec((3, BN), lambda i: (0, i)),
        pl.BlockSpec((3, 1), lambda i: (0, 0)),
    ],
    out_specs=pl.BlockSpec((BN, 1), lambda i: (i, 0)),
    out_shape=jax.ShapeDtypeStruct((NPAD, 1), f32),
)


def kernel(x, edge_index_v1, edge_index_v2, edge_index_v3,
           edge_weight_v1, edge_weight_v2, edge_weight_v3,
           W1_v1, b1_v1, W2_v1, b2_v1,
           W1_v2, b1_v2, W2_v2, b2_v2,
           W1_v3, b1_v3, W2_v3, b2_v3):
    row1, col1 = edge_index_v1[0], edge_index_v1[1]
    row2, col2 = edge_index_v2[0], edge_index_v2[1]
    row3, col3 = edge_index_v3[0], edge_index_v3[1]

    P = _hist(col1, col2, col3, edge_weight_v1, edge_weight_v2,
              edge_weight_v3).reshape(2, 6, NPAD)

    xpad = jnp.pad(x, ((0, NPAD - N), (0, 0)))
    W1all = jnp.concatenate([W1_v1, W1_v2, W1_v3], axis=1)
    gt1, gt2, gt3, d1, d2 = _tcb(xpad, W1all, P)

    accp = _agg(gt1, gt2, gt3, row1, row2, row3, col1, col2, col3,
                edge_weight_v1, edge_weight_v2, edge_weight_v3)

    b1all = jnp.stack([b1_v1, b1_v2, b1_v3])
    W2all = jnp.concatenate([W2_v1, W2_v2, W2_v3], axis=1)
    feat, g2 = _tcd(accp, gt1, gt2, gt3, d1, d2, b1all, W2all)

    a2p = _l2(g2.reshape(-1), row1, row2, row3,
              col1, col2, col3).reshape(2, 3, NPAD)

    b2all = jnp.stack([b2_v1, b2_v2, b2_v3])
    xout = _tcf(a2p, g2, d2, b2all)

    return (xout[:N, 0], feat[:N])


# speedup vs baseline: 1.2049x; 1.2049x over previous
import jax
import jax.numpy as jnp
from jax import lax
from jax.experimental import pallas as pl
from jax.experimental.pallas import tpu as pltpu
from jax.experimental.pallas import tpu_sc as plsc

N = 10000
E = 320000
D = 128
H = 64
NPAD = 10240
CHUNK = 128
NCHUNK = E // CHUNK
CH_PER_CORE = NCHUNK // 2
CH_BASE = CH_PER_CORE // 16
CH_REM = CH_PER_CORE - CH_BASE * 16
RPT = NPAD // 16
HOPS = RPT // CHUNK
BN = 512
GRID = NPAD // BN

f32 = jnp.float32
i32 = jnp.int32

_mesh = plsc.VectorSubcoreMesh(core_axis_name="c", subcore_axis_name="s")
_sc_params = pltpu.CompilerParams(use_tc_tiling_on_sc=False)


def _chunk_range(c, s):
    ntch = CH_BASE + jnp.where(s < CH_REM, 1, 0)
    start = c * CH_PER_CORE + s * CH_BASE + jnp.minimum(s, CH_REM)
    return start, ntch


def _hist_body(col1, col2, col3, ew1, ew2, ew3, P,
               colbuf0, colbuf1, valbuf0, valbuf1, onesbuf, tmp, zbuf,
               wd0, cn0, wd1, cn1, wd2, cn2,
               ldsem0, ldsem1, sca0, sca1, scb0, scb1):
    c = lax.axis_index("c")
    s = lax.axis_index("s")
    zero16 = jnp.zeros((16,), f32)
    one16 = jnp.ones((16,), f32)
    colbuf = (colbuf0, colbuf1)
    valbuf = (valbuf0, valbuf1)
    ldsem = (ldsem0, ldsem1)
    sca = (sca0, sca1)
    scb = (scb0, scb1)
    for i in range(RPT // 16):
        zbuf[pl.ds(i * 16, 16)] = zero16
    for i in range(CHUNK // 16):
        onesbuf[pl.ds(i * 16, 16)] = one16
    sl = pl.ds(s * RPT, RPT)
    for arr in (wd0, cn0, wd1, cn1, wd2, cn2):
        pltpu.sync_copy(zbuf, arr.at[sl])
    plsc.subcore_barrier()
    start, ntch = _chunk_range(c, s)
    end = start + ntch
    for colr, ewr, wd, cn in ((col1, ew1, wd0, cn0),
                              (col2, ew2, wd1, cn1),
                              (col3, ew3, wd2, cn2)):
        def fire_loads(cc, b, colr=colr, ewr=ewr):
            base = cc * CHUNK
            pltpu.async_copy(colr.at[pl.ds(base, CHUNK)], colbuf[b], ldsem[b])
            pltpu.async_copy(ewr.at[pl.ds(base, CHUNK)], valbuf[b], ldsem[b])

        def wait_scatters(b, wd=wd, cn=cn):
            pltpu.make_async_copy(valbuf[b], wd.at[colbuf[b]], sca[b]).wait()
            pltpu.make_async_copy(onesbuf, cn.at[colbuf[b]], scb[b]).wait()

        def proc(cc, b, colr=colr, ewr=ewr, wd=wd, cn=cn):
            @pl.when(cc >= start + 1)
            def _():
                wait_scatters(1 - b)

            @pl.when(cc + 1 < end)
            def _():
                fire_loads(cc + 1, 1 - b)
            pltpu.make_async_copy(colr.at[pl.ds(0, CHUNK)], colbuf[b],
                                  ldsem[b]).wait()
            pltpu.make_async_copy(ewr.at[pl.ds(0, CHUNK)], valbuf[b],
                                  ldsem[b]).wait()
            pltpu.async_copy(valbuf[b], wd.at[colbuf[b]], sca[b], add=True)
            pltpu.async_copy(onesbuf, cn.at[colbuf[b]], scb[b], add=True)

        fire_loads(start, 0)

        def pair(k, carry):
            proc(start + 2 * k, 0)
            proc(start + 2 * k + 1, 1)
            return carry
        lax.fori_loop(0, ntch // 2, pair, 0)

        @pl.when(ntch % 2 == 1)
        def _(wait_scatters=wait_scatters, proc=proc):
            proc(end - 1, 0)
            wait_scatters(0)

        @pl.when(ntch % 2 == 0)
        def _(wait_scatters=wait_scatters):
            wait_scatters(1)
    plsc.subcore_barrier()
    for k, arr in enumerate((wd0, wd1, wd2, cn0, cn1, cn2)):
        pltpu.sync_copy(arr.at[sl], tmp)
        pltpu.sync_copy(tmp, P.at[pl.ds((c * 6 + k) * NPAD + s * RPT, RPT)])


_hist = pl.kernel(
    _hist_body,
    out_type=jax.ShapeDtypeStruct((12 * NPAD,), f32),
    mesh=_mesh,
    compiler_params=_sc_params,
    scratch_types=[
        pltpu.VMEM((CHUNK,), i32),
        pltpu.VMEM((CHUNK,), i32),
        pltpu.VMEM((CHUNK,), f32),
        pltpu.VMEM((CHUNK,), f32),
        pltpu.VMEM((CHUNK,), f32),
        pltpu.VMEM((RPT,), f32),
        pltpu.VMEM((RPT,), f32),
    ] + [pltpu.VMEM_SHARED((NPAD,), f32)] * 6
      + [pltpu.SemaphoreType.DMA] * 6,
)


def _agg_body(g1, g2, g3, row1, row2, row3, col1, col2, col3,
              ew1, ew2, ew3, accp,
              rowbuf0, rowbuf1, rowbuf2, colbuf0, colbuf1, colbuf2,
              ewbuf0, ewbuf1, ewbuf2, rows0, rows1, rows2, zrows, table, acc,
              ldsem0, ldsem1, ldsem2, gsem0, gsem1, gsem2,
              scsem0, scsem1, scsem2):
    c = lax.axis_index("c")
    s = lax.axis_index("s")
    zero16 = jnp.zeros((16,), f32)
    rowbuf = (rowbuf0, rowbuf1, rowbuf2)
    colbuf = (colbuf0, colbuf1, colbuf2)
    ewbuf = (ewbuf0, ewbuf1, ewbuf2)
    rows = (rows0, rows1, rows2)
    ldsem = (ldsem0, ldsem1, ldsem2)
    gsem = (gsem0, gsem1, gsem2)
    scsem = (scsem0, scsem1, scsem2)

    for r in range(CHUNK):
        for q in range(H // 16):
            zrows[r, pl.ds(q * 16, 16)] = zero16
    start, ntch = _chunk_range(c, s)
    end = start + ntch
    for v, (gv, rowr, colr, ewr) in enumerate(((g1, row1, col1, ew1),
                                               (g2, row2, col2, ew2),
                                               (g3, row3, col3, ew3))):
        for hp in range(HOPS):
            r0 = s * RPT + hp * CHUNK
            pltpu.sync_copy(gv.at[pl.ds(r0, CHUNK)], rows0)
            pltpu.sync_copy(rows0, table.at[pl.ds(r0, CHUNK)])
            pltpu.sync_copy(zrows, acc.at[pl.ds(r0, CHUNK)])
        plsc.subcore_barrier()

        def fire_loads(cc, b, rowr=rowr, colr=colr, ewr=ewr):
            base = cc * CHUNK
            pltpu.async_copy(rowr.at[pl.ds(base, CHUNK)], rowbuf[b], ldsem[b])
            pltpu.async_copy(colr.at[pl.ds(base, CHUNK)], colbuf[b], ldsem[b])
            pltpu.async_copy(ewr.at[pl.ds(base, CHUNK)], ewbuf[b], ldsem[b])

        def wait_loads(b, rowr=rowr, colr=colr, ewr=ewr):
            pltpu.make_async_copy(rowr.at[pl.ds(0, CHUNK)], rowbuf[b],
                                  ldsem[b]).wait()
            pltpu.make_async_copy(colr.at[pl.ds(0, CHUNK)], colbuf[b],
                                  ldsem[b]).wait()
            pltpu.make_async_copy(ewr.at[pl.ds(0, CHUNK)], ewbuf[b],
                                  ldsem[b]).wait()

        def fire_gather(b):
            pltpu.async_copy(table.at[rowbuf[b]], rows[b], gsem[b])

        def wait_gather(b):
            pltpu.make_async_copy(table.at[rowbuf[b]], rows[b],
                                  gsem[b]).wait()

        def wait_scatter(b):
            pltpu.make_async_copy(rows[b], acc.at[colbuf[b]],
                                  scsem[b]).wait()

        def proc(cc, b, fire_loads=fire_loads, wait_loads=wait_loads,
                 fire_gather=fire_gather, wait_gather=wait_gather,
                 wait_scatter=wait_scatter):
            b1 = (b + 1) % 3
            b2 = (b + 2) % 3

            @pl.when(cc >= start + 1)
            def _():
                wait_scatter(b2)

            @pl.when(cc + 2 < end)
            def _():
                fire_loads(cc + 2, b2)
            wait_gather(b)

            @pl.when(cc + 1 < end)
            def _():
                wait_loads(b1)
                fire_gather(b1)

            def scale(gg, cr):
                vew = ewbuf[b][pl.ds(gg * 16, 16)]
                for eo in range(16):
                    w = vew.at[jnp.full((16,), eo, i32)].get(
                        mode="promise_in_bounds")
                    e = gg * 16 + eo
                    for q in range(H // 16):
                        rows[b][e, pl.ds(q * 16, 16)] = (
                            rows[b][e, pl.ds(q * 16, 16)] * w)
                return cr
            lax.fori_loop(0, CHUNK // 16, scale, 0)
            pltpu.async_copy(rows[b], acc.at[colbuf[b]], scsem[b], add=True)

        fire_loads(start, 0)
        fire_loads(start + 1, 1)
        wait_loads(0)
        fire_gather(0)

        def triple(k, carry):
            c0 = start + 3 * k
            proc(c0, 0)
            proc(c0 + 1, 1)
            proc(c0 + 2, 2)
            return carry
        lax.fori_loop(0, ntch // 3, triple, 0)

        @pl.when(ntch % 3 == 1)
        def _(proc=proc, wait_scatter=wait_scatter):
            proc(end - 1, 0)
            wait_scatter(0)

        @pl.when(ntch % 3 == 0)
        def _(wait_scatter=wait_scatter):
            wait_scatter(2)
        plsc.subcore_barrier()
        for hp in range(HOPS):
            r0 = s * RPT + hp * CHUNK
            pltpu.sync_copy(acc.at[pl.ds(r0, CHUNK)], rows0)
            pltpu.sync_copy(rows0, accp.at[c, v, pl.ds(r0, CHUNK)])
        plsc.subcore_barrier()


_agg = pl.kernel(
    _agg_body,
    out_type=jax.ShapeDtypeStruct((2, 3, NPAD, H), f32),
    mesh=_mesh,
    compiler_params=_sc_params,
    scratch_types=(
        [pltpu.VMEM((CHUNK,), i32)] * 6
        + [pltpu.VMEM((CHUNK,), f32)] * 3
        + [pltpu.VMEM((CHUNK, H), f32)] * 4
        + [pltpu.VMEM_SHARED((NPAD, H), f32)] * 2
        + [pltpu.SemaphoreType.DMA] * 9
    ),
)


def _l2_body(g2, row1, row2, row3, col1, col2, col3, a2p,
             rowbuf0, rowbuf1, colbuf0, colbuf1, valbuf0, valbuf1,
             tmp, zbuf, t0, a0, t1, a1, t2, a2,
             ldsem0, ldsem1, gsem, scsem0, scsem1):
    c = lax.axis_index("c")
    s = lax.axis_index("s")
    zero16 = jnp.zeros((16,), f32)
    rowbuf = (rowbuf0, rowbuf1)
    colbuf = (colbuf0, colbuf1)
    valbuf = (valbuf0, valbuf1)
    ldsem = (ldsem0, ldsem1)
    scsem = (scsem0, scsem1)
    for i in range(RPT // 16):
        zbuf[pl.ds(i * 16, 16)] = zero16
    sl = pl.ds(s * RPT, RPT)
    for v, (tv, av) in enumerate(((t0, a0), (t1, a1), (t2, a2))):
        pltpu.sync_copy(g2.at[pl.ds(v * NPAD + s * RPT, RPT)], tmp)
        pltpu.sync_copy(tmp, tv.at[sl])
        pltpu.sync_copy(zbuf, av.at[sl])
    plsc.subcore_barrier()
    start, ntch = _chunk_range(c, s)
    end = start + ntch
    for (rowr, colr, tv, av) in ((row1, col1, t0, a0),
                                 (row2, col2, t1, a1),
                                 (row3, col3, t2, a2)):
        def fire_loads(cc, b, rowr=rowr, colr=colr):
            base = cc * CHUNK
            pltpu.async_copy(rowr.at[pl.ds(base, CHUNK)], rowbuf[b], ldsem[b])
            pltpu.async_copy(colr.at[pl.ds(base, CHUNK)], colbuf[b], ldsem[b])

        def wait_scatter(b, av=av):
            pltpu.make_async_copy(valbuf[b], av.at[colbuf[b]],
                                  scsem[b]).wait()

        def proc(cc, b, rowr=rowr, colr=colr, tv=tv, av=av):
            @pl.when(cc >= start + 1)
            def _():
                wait_scatter(1 - b)

            @pl.when(cc + 1 < end)
            def _():
                fire_loads(cc + 1, 1 - b)
            pltpu.make_async_copy(rowr.at[pl.ds(0, CHUNK)], rowbuf[b],
                                  ldsem[b]).wait()
            pltpu.make_async_copy(colr.at[pl.ds(0, CHUNK)], colbuf[b],
                                  ldsem[b]).wait()
            pltpu.async_copy(tv.at[rowbuf[b]], valbuf[b], gsem).wait()
            pltpu.async_copy(valbuf[b], av.at[colbuf[b]], scsem[b], add=True)

        fire_loads(start, 0)

        def pair(k, carry):
            proc(start + 2 * k, 0)
            proc(start + 2 * k + 1, 1)
            return carry
        lax.fori_loop(0, ntch // 2, pair, 0)

        @pl.when(ntch % 2 == 1)
        def _(wait_scatter=wait_scatter, proc=proc):
            proc(end - 1, 0)
            wait_scatter(0)

        @pl.when(ntch % 2 == 0)
        def _(wait_scatter=wait_scatter):
            wait_scatter(1)
    plsc.subcore_barrier()
    for v, av in enumerate((a0, a1, a2)):
        pltpu.sync_copy(av.at[sl], tmp)
        pltpu.sync_copy(tmp, a2p.at[pl.ds((c * 3 + v) * NPAD + s * RPT, RPT)])


_l2 = pl.kernel(
    _l2_body,
    out_type=jax.ShapeDtypeStruct((6 * NPAD,), f32),
    mesh=_mesh,
    compiler_params=_sc_params,
    scratch_types=[
        pltpu.VMEM((CHUNK,), i32),
        pltpu.VMEM((CHUNK,), i32),
        pltpu.VMEM((CHUNK,), i32),
        pltpu.VMEM((CHUNK,), i32),
        pltpu.VMEM((CHUNK,), f32),
        pltpu.VMEM((CHUNK,), f32),
        pltpu.VMEM((RPT,), f32),
        pltpu.VMEM((RPT,), f32),
    ] + [pltpu.VMEM_SHARED((NPAD,), f32)] * 6
      + [pltpu.SemaphoreType.DMA] * 5,
)


def _tcb_body(x_ref, w_ref, p_ref, g1_ref, g2_ref, g3_ref, d1_ref, d2_ref):
    h = jnp.dot(x_ref[...], w_ref[...], preferred_element_type=f32)
    p = p_ref[...]
    for v, g_ref in enumerate((g1_ref, g2_ref, g3_ref)):
        d1 = lax.rsqrt(p[0, v, :] + p[1, v, :] + 1.0)
        d2 = lax.rsqrt(p[0, 3 + v, :] + p[1, 3 + v, :] + 1.0)
        d1_ref[v, :] = d1
        d2_ref[v, :] = d2
        g_ref[...] = h[:, v * H:(v + 1) * H] * d1[:, None]


_tcb = pl.pallas_call(
    _tcb_body,
    grid=(GRID,),
    in_specs=[
        pl.BlockSpec((BN, D), lambda i: (i, 0)),
        pl.BlockSpec((D, 3 * H), lambda i: (0, 0)),
        pl.BlockSpec((2, 6, BN), lambda i: (0, 0, i)),
    ],
    out_specs=[
        pl.BlockSpec((BN, H), lambda i: (i, 0)),
        pl.BlockSpec((BN, H), lambda i: (i, 0)),
        pl.BlockSpec((BN, H), lambda i: (i, 0)),
        pl.BlockSpec((3, BN), lambda i: (0, i)),
        pl.BlockSpec((3, BN), lambda i: (0, i)),
    ],
    out_shape=[
        jax.ShapeDtypeStruct((NPAD, H), f32),
        jax.ShapeDtypeStruct((NPAD, H), f32),
        jax.ShapeDtypeStruct((NPAD, H), f32),
        jax.ShapeDtypeStruct((3, NPAD), f32),
        jax.ShapeDtypeStruct((3, NPAD), f32),
    ],
)


def _tcd_body(ap_ref, g1_ref, g2t_ref, g3_ref, d1_ref, d2_ref, b1_ref, w2_ref,
              feat_ref, g2_ref):
    feat = jnp.zeros((BN, H), f32)
    w2 = w2_ref[...]
    for v, g_ref in enumerate((g1_ref, g2t_ref, g3_ref)):
        acc = ap_ref[0, v] + ap_ref[1, v]
        xv = jnp.maximum(
            d1_ref[v][:, None] * (acc + g_ref[...]) + b1_ref[v][None, :], 0.0)
        feat = feat + xv
        h2 = jnp.dot(xv, w2[:, v:v + 1], preferred_element_type=f32)[:, 0]
        g2_ref[v, :] = d2_ref[v] * h2
    feat_ref[...] = feat


_tcd = pl.pallas_call(
    _tcd_body,
    grid=(GRID,),
    in_specs=[
        pl.BlockSpec((2, 3, BN, H), lambda i: (0, 0, i, 0)),
        pl.BlockSpec((BN, H), lambda i: (i, 0)),
        pl.BlockSpec((BN, H), lambda i: (i, 0)),
        pl.BlockSpec((BN, H), lambda i: (i, 0)),
        pl.BlockSpec((3, BN), lambda i: (0, i)),
        pl.BlockSpec((3, BN), lambda i: (0, i)),
        pl.BlockSpec((3, H), lambda i: (0, 0)),
        pl.BlockSpec((H, 3), lambda i: (0, 0)),
    ],
    out_specs=[
        pl.BlockSpec((BN, H), lambda i: (i, 0)),
        pl.BlockSpec((3, BN), lambda i: (0, i)),
    ],
    out_shape=[
        jax.ShapeDtypeStruct((NPAD, H), f32),
        jax.ShapeDtypeStruct((3, NPAD), f32),
    ],
)


def _tcf_body(a2_ref, g2_ref, d2_ref, b2_ref, out_ref):
    b2 = b2_ref[...]
    tot = jnp.zeros((BN,), f32)
    for v in range(3):
        tot = tot + d2_ref[v] * (a2_ref[0, v] + a2_ref[1, v] + g2_ref[v]) \
            + b2[v, 0]
    out_ref[...] = tot[:, None]


_tcf = pl.pallas_call(
    _tcf_body,
    grid=(GRID,),
    in_specs=[
        pl.BlockSpec((2, 3, BN), lambda i: (0, 0, i)),
        pl.BlockSpec((3, BN), lambda i: (0, i)),
        pl.BlockSpec((3, BN), lambda i: (0, i)),
        pl.BlockSpec((3, 1), lambda i: (0, 0)),
    ],
    out_specs=pl.BlockSpec((BN, 1), lambda i: (i, 0)),
    out_shape=jax.ShapeDtypeStruct((NPAD, 1), f32),
)


def kernel(x, edge_index_v1, edge_index_v2, edge_index_v3,
           edge_weight_v1, edge_weight_v2, edge_weight_v3,
           W1_v1, b1_v1, W2_v1, b2_v1,
           W1_v2, b1_v2, W2_v2, b2_v2,
           W1_v3, b1_v3, W2_v3, b2_v3):
    row1, col1 = edge_index_v1[0], edge_index_v1[1]
    row2, col2 = edge_index_v2[0], edge_index_v2[1]
    row3, col3 = edge_index_v3[0], edge_index_v3[1]

    P = _hist(col1, col2, col3, edge_weight_v1, edge_weight_v2,
              edge_weight_v3).reshape(2, 6, NPAD)

    xpad = jnp.pad(x, ((0, NPAD - N), (0, 0)))
    W1all = jnp.concatenate([W1_v1, W1_v2, W1_v3], axis=1)
    gt1, gt2, gt3, d1, d2 = _tcb(xpad, W1all, P)

    accp = _agg(gt1, gt2, gt3, row1, row2, row3, col1, col2, col3,
                edge_weight_v1, edge_weight_v2, edge_weight_v3)

    b1all = jnp.stack([b1_v1, b1_v2, b1_v3])
    W2all = jnp.concatenate([W2_v1, W2_v2, W2_v3], axis=1)
    feat, g2 = _tcd(accp, gt1, gt2, gt3, d1, d2, b1all, W2all)

    a2p = _l2(g2.reshape(-1), row1, row2, row3,
              col1, col2, col3).reshape(2, 3, NPAD)

    b2all = jnp.stack([b2_v1, b2_v2, b2_v3])
    xout = _tcf(a2p, g2, d2, b2all)

    return (xout[:N, 0], feat[:N])
